# baseline (speedup 1.0000x reference)
import functools

import jax
import jax.numpy as jnp
from jax import lax
from jax.experimental import pallas as pl
from jax.experimental.pallas import tpu as pltpu
from jax.experimental.pallas import tpu_sc as plsc

N = 10000
D = 128
HD = D // 2
E = 320000

NUM_CORES = 2
NUM_SUBCORES = 16
NUM_TILES = NUM_CORES * NUM_SUBCORES

N_PAD = 10240
EDGES_PER_TILE = E // NUM_TILES
CHUNK = 128
AGGW = 200
AGG_STEPS = EDGES_PER_TILE // AGGW
ROWS_PER_SUBCORE = N_PAD // NUM_SUBCORES

BLK = 2560
GRID = N_PAD // BLK


def _mesh():
    return plsc.VectorSubcoreMesh(core_axis_name="c", subcore_axis_name="s")


def _zero_vmem_2d(ref, rows, cols, dtype=jnp.float32):
    lanes = 32 if dtype == jnp.bfloat16 else 16

    @pl.loop(0, rows)
    def _(r):
        @pl.loop(0, cols, step=lanes)
        def _(j):
            ref[r, pl.ds(j, lanes)] = jnp.zeros((lanes,), dtype)


@jax.jit
def _sc_degree(dst2d, ones_rows):
    @functools.partial(
        pl.kernel,
        out_type=jax.ShapeDtypeStruct((NUM_CORES, N_PAD, 16), jnp.float32),
        mesh=_mesh(),
        compiler_params=pltpu.CompilerParams(use_tc_tiling_on_sc=False),
        scratch_types=[
            pltpu.VMEM((EDGES_PER_TILE,), jnp.int32),
            pltpu.VMEM((AGGW, 16), jnp.float32),
            pltpu.VMEM((CHUNK, 16), jnp.float32),
            pltpu.VMEM_SHARED((N_PAD, 16), jnp.float32),
        ],
    )
    def deg_kernel(dst_hbm, ones_hbm, out_hbm, idx_v, ones_v, zbuf_v, acc_sh):
        c = lax.axis_index("c")
        s = lax.axis_index("s")
        wid = c * NUM_SUBCORES + s

        _zero_vmem_2d(zbuf_v, CHUNK, 16)
        base_row = s * ROWS_PER_SUBCORE

        @pl.loop(0, ROWS_PER_SUBCORE, step=CHUNK)
        def _(j):
            pltpu.sync_copy(zbuf_v, acc_sh.at[pl.ds(base_row + j, CHUNK)])

        pltpu.sync_copy(ones_hbm, ones_v)
        pltpu.sync_copy(
            dst_hbm.at[pl.ds(wid * EDGES_PER_TILE, EDGES_PER_TILE)], idx_v)
        plsc.subcore_barrier()

        @pl.loop(0, AGG_STEPS)
        def _(k):
            pltpu.sync_copy(ones_v, acc_sh.at[idx_v.at[pl.ds(k * AGGW, AGGW)]],
                            add=True)

        plsc.subcore_barrier()
        pltpu.sync_copy(
            acc_sh.at[pl.ds(base_row, ROWS_PER_SUBCORE)],
            out_hbm.at[c, pl.ds(base_row, ROWS_PER_SUBCORE)],
        )

    return deg_kernel(dst2d, ones_rows)


@jax.jit
def _sc_aggregate(y, src2d, dst2d):
    @functools.partial(
        pl.kernel,
        out_type=jax.ShapeDtypeStruct((NUM_CORES, N_PAD, D), jnp.bfloat16),
        mesh=_mesh(),
        compiler_params=pltpu.CompilerParams(use_tc_tiling_on_sc=False),
        scratch_types=[
            pltpu.VMEM((EDGES_PER_TILE,), jnp.int32),
            pltpu.VMEM((EDGES_PER_TILE,), jnp.int32),
            pltpu.VMEM((AGGW, D), jnp.bfloat16),
            pltpu.VMEM((AGGW, D), jnp.bfloat16),
            pltpu.VMEM((CHUNK, D), jnp.bfloat16),
            pltpu.VMEM_SHARED((N_PAD, D), jnp.bfloat16),
            pltpu.SemaphoreType.DMA,
            pltpu.SemaphoreType.DMA,
        ],
    )
    def agg_kernel(y_hbm, src_hbm, dst_hbm, out_hbm,
                   isrc_v, idst_v, rows_a, rows_b, zbuf_v, acc_sh,
                   sem_a, sem_b):
        c = lax.axis_index("c")
        s = lax.axis_index("s")
        wid = c * NUM_SUBCORES + s
        base_row = s * ROWS_PER_SUBCORE

        _zero_vmem_2d(zbuf_v, CHUNK, D, jnp.bfloat16)

        ebase = wid * EDGES_PER_TILE
        pltpu.sync_copy(src_hbm.at[pl.ds(ebase, EDGES_PER_TILE)], isrc_v)
        pltpu.sync_copy(dst_hbm.at[pl.ds(ebase, EDGES_PER_TILE)], idst_v)

        @pl.loop(0, ROWS_PER_SUBCORE, step=CHUNK)
        def _(j):
            pltpu.sync_copy(zbuf_v, acc_sh.at[pl.ds(base_row + j, CHUNK)])

        plsc.subcore_barrier()

        pltpu.make_async_copy(
            y_hbm.at[isrc_v.at[pl.ds(0, AGGW)]], rows_a, sem_a).start()
        pltpu.make_async_copy(
            y_hbm.at[isrc_v.at[pl.ds(AGGW, AGGW)]], rows_b, sem_b).start()

        @pl.loop(0, AGG_STEPS, step=2)
        def _(k):
            for off, buf, sem in ((0, rows_a, sem_a), (1, rows_b, sem_b)):
                kk = (k + off) * AGGW
                pltpu.make_async_copy(
                    y_hbm.at[isrc_v.at[pl.ds(kk, AGGW)]], buf, sem).wait()
                pltpu.sync_copy(buf, acc_sh.at[idst_v.at[pl.ds(kk, AGGW)]],
                                add=True)

                @pl.when(kk + 2 * AGGW < EDGES_PER_TILE)
                def _():
                    pltpu.make_async_copy(
                        y_hbm.at[isrc_v.at[pl.ds(kk + 2 * AGGW, AGGW)]], buf,
                        sem).start()

        plsc.subcore_barrier()
        pltpu.sync_copy(
            acc_sh.at[pl.ds(base_row, ROWS_PER_SUBCORE)],
            out_hbm.at[c, pl.ds(base_row, ROWS_PER_SUBCORE)],
        )

    return agg_kernel(y, src2d, dst2d)


def _dinv_col(deg_ref, blk_i):
    deg = deg_ref[0] + deg_ref[1] + 1.0
    rowid = blk_i * BLK + lax.broadcasted_iota(jnp.int32, (BLK, 16), 0)
    dinv = jnp.where(rowid < N, lax.rsqrt(deg), 0.0)
    return dinv[:, 0:1]


def _xp_gram_body(x_ref, w_ref, xp_ref, g_ref):
    xp = lax.dot_general(x_ref[...], w_ref[...],
                         dimension_numbers=(((1,), (1,)), ((), ())),
                         preferred_element_type=jnp.float32,
                         precision=lax.Precision.DEFAULT)
    xp_ref[...] = xp

    @pl.when(pl.program_id(0) == 0)
    def _():
        g_ref[...] = jnp.zeros((D, D), jnp.float32)

    g_ref[...] += lax.dot_general(xp, xp,
                                  dimension_numbers=(((0,), (0,)), ((), ())),
                                  preferred_element_type=jnp.float32,
                                  precision=lax.Precision.DEFAULT)


@jax.jit
def _tc_xp_gram(x_pad, W1):
    return pl.pallas_call(
        _xp_gram_body,
        grid=(GRID,),
        in_specs=[
            pl.BlockSpec((BLK, D), lambda i: (i, 0)),
            pl.BlockSpec((D, D), lambda i: (0, 0)),
        ],
        out_specs=[
            pl.BlockSpec((BLK, D), lambda i: (i, 0)),
            pl.BlockSpec((D, D), lambda i: (0, 0)),
        ],
        out_shape=[
            jax.ShapeDtypeStruct((N_PAD, D), jnp.float32),
            jax.ShapeDtypeStruct((D, D), jnp.float32),
        ],
    )(x_pad, W1)


def _y_body(xp_ref, deg_ref, y_ref):
    y_ref[...] = (xp_ref[...] * _dinv_col(deg_ref, pl.program_id(0))
                  ).astype(jnp.bfloat16)


@jax.jit
def _tc_y(xp, degp):
    return pl.pallas_call(
        _y_body,
        grid=(GRID,),
        in_specs=[
            pl.BlockSpec((BLK, D), lambda i: (i, 0)),
            pl.BlockSpec((NUM_CORES, BLK, 16), lambda i: (0, i, 0)),
        ],
        out_specs=pl.BlockSpec((BLK, D), lambda i: (i, 0)),
        out_shape=jax.ShapeDtypeStruct((N_PAD, D), jnp.bfloat16),
    )(xp, degp)


def _gram_body(xp_ref, g_ref):
    @pl.when(pl.program_id(0) == 0)
    def _():
        g_ref[...] = jnp.zeros((D, D), jnp.float32)

    xb = xp_ref[...]
    g_ref[...] += lax.dot_general(xb, xb,
                                  dimension_numbers=(((0,), (0,)), ((), ())),
                                  preferred_element_type=jnp.float32,
                                  precision=lax.Precision.DEFAULT)


@jax.jit
def _tc_gram(xp):
    return pl.pallas_call(
        _gram_body,
        grid=(GRID,),
        in_specs=[pl.BlockSpec((BLK, D), lambda i: (i, 0))],
        out_specs=pl.BlockSpec((D, D), lambda i: (0, 0)),
        out_shape=jax.ShapeDtypeStruct((D, D), jnp.float32),
    )(xp)


def _combine_core(xp_ref, y_ref, parts_ref, deg_ref, gmat_ref, gate_ref,
                  b_ref, i):
    zeta, eta, theta = gate_ref[0], gate_ref[1], gate_ref[2]
    xp = xp_ref[...]
    dcol = _dinv_col(deg_ref, i)
    psum = (parts_ref[0].astype(jnp.float32)
            + parts_ref[1].astype(jnp.float32))
    agg = dcol * (psum + y_ref[...].astype(jnp.float32))
    red = lax.dot_general(xp, gmat_ref[...],
                          dimension_numbers=(((1,), (0,)), ((), ())),
                          preferred_element_type=jnp.float32,
                          precision=lax.Precision.DEFAULT)
    return zeta * xp + eta * agg - theta * red + b_ref[...], dcol


def _mid_body(xp_ref, y_ref, parts_ref, deg_ref, gmat_ref, gate_ref, b_ref,
              gam_ref, bet_ref, mu_ref, var_ref, wn_ref, xpn_ref, yn_ref):
    i = pl.program_id(0)
    out, dcol = _combine_core(xp_ref, y_ref, parts_ref, deg_ref, gmat_ref,
                              gate_ref, b_ref, i)
    h = (out - mu_ref[...]) * lax.rsqrt(var_ref[...] + 1e-5) * gam_ref[...] \
        + bet_ref[...]
    h = jnp.maximum(h, 0.0)
    rowid = i * BLK + lax.broadcasted_iota(jnp.int32, (BLK, D), 0)
    h = jnp.where(rowid < N, h, 0.0)
    xpn = lax.dot_general(h, wn_ref[...],
                          dimension_numbers=(((1,), (1,)), ((), ())),
                          preferred_element_type=jnp.float32,
                          precision=lax.Precision.DEFAULT)
    xpn_ref[...] = xpn
    yn_ref[...] = (xpn * dcol).astype(jnp.bfloat16)


@jax.jit
def _tc_mid(xp, y3, parts, degp, gmat, gates, bvec, gam, bet, mu, var, Wn):
    vec = lambda: pl.BlockSpec((1, D), lambda i: (0, 0))
    return pl.pallas_call(
        _mid_body,
        grid=(GRID,),
        in_specs=[
            pl.BlockSpec((BLK, D), lambda i: (i, 0)),
            pl.BlockSpec((BLK, D), lambda i: (i, 0)),
            pl.BlockSpec((NUM_CORES, BLK, D), lambda i: (0, i, 0)),
            pl.BlockSpec((NUM_CORES, BLK, 16), lambda i: (0, i, 0)),
            pl.BlockSpec((D, D), lambda i: (0, 0)),
            pl.BlockSpec(memory_space=pltpu.SMEM),
            vec(), vec(), vec(), vec(), vec(),
            pl.BlockSpec((D, D), lambda i: (0, 0)),
        ],
        out_specs=[
            pl.BlockSpec((BLK, D), lambda i: (i, 0)),
            pl.BlockSpec((BLK, D), lambda i: (i, 0)),
        ],
        out_shape=[
            jax.ShapeDtypeStruct((N_PAD, D), jnp.float32),
            jax.ShapeDtypeStruct((N_PAD, D), jnp.bfloat16),
        ],
    )(xp, y3, parts, degp, gmat, gates, bvec, gam, bet, mu, var, Wn)


def _last_body(xp_ref, y_ref, parts_ref, deg_ref, gmat_ref, gate_ref, b_ref,
               out_ref):
    i = pl.program_id(0)
    out, _ = _combine_core(xp_ref, y_ref, parts_ref, deg_ref, gmat_ref,
                           gate_ref, b_ref, i)
    out_ref[...] = out


@jax.jit
def _tc_last(xp, y3, parts, degp, gmat, gates, bvec):
    return pl.pallas_call(
        _last_body,
        grid=(GRID,),
        in_specs=[
            pl.BlockSpec((BLK, D), lambda i: (i, 0)),
            pl.BlockSpec((BLK, D), lambda i: (i, 0)),
            pl.BlockSpec((NUM_CORES, BLK, D), lambda i: (0, i, 0)),
            pl.BlockSpec((NUM_CORES, BLK, 16), lambda i: (0, i, 0)),
            pl.BlockSpec((D, D), lambda i: (0, 0)),
            pl.BlockSpec(memory_space=pltpu.SMEM),
            pl.BlockSpec((1, D), lambda i: (0, 0)),
        ],
        out_specs=pl.BlockSpec((BLK, D), lambda i: (i, 0)),
        out_shape=jax.ShapeDtypeStruct((N_PAD, D), jnp.float32),
    )(xp, y3, parts, degp, gmat, gates, bvec)


def kernel(x, edge_index, W1, b1, g1, W2, b2, g2, W3, b3, g3,
           bn1_gamma, bn1_beta, bn1_mean, bn1_var,
           bn2_gamma, bn2_beta, bn2_mean, bn2_var):
    src_flat = edge_index[0]
    dst_flat = edge_index[1]
    x_pad = jnp.concatenate(
        [x, jnp.zeros((N_PAD - N, D), dtype=jnp.float32)], axis=0)
    ones_rows = jnp.ones((AGGW, 16), jnp.float32)

    degp = _sc_degree(dst_flat, ones_rows)
    xp, gmat1 = _tc_xp_gram(x_pad, W1)
    y3 = _tc_y(xp, degp)

    bn = [(bn1_gamma, bn1_beta, bn1_mean, bn1_var),
          (bn2_gamma, bn2_beta, bn2_mean, bn2_var)]
    row = lambda v: v.reshape(1, D)

    for lyr, (gate, bvec, Wn) in enumerate(((g1, b1, W2), (g2, b2, W3))):
        parts = _sc_aggregate(y3, src_flat, dst_flat)
        gmat = gmat1 if lyr == 0 else _tc_gram(xp)
        gam, bet, mu, var = bn[lyr]
        xp, y3 = _tc_mid(xp, y3, parts, degp, gmat, jax.nn.softmax(gate),
                         row(bvec), row(gam), row(bet), row(mu), row(var), Wn)

    parts = _sc_aggregate(y3, src_flat, dst_flat)
    gmat = _tc_gram(xp)
    out = _tc_last(xp, y3, parts, degp, gmat, jax.nn.softmax(g3), row(b3))
    return out[:N]


# speedup vs baseline: 1.0084x; 1.0084x over previous
import functools

import jax
import jax.numpy as jnp
from jax import lax
from jax.experimental import pallas as pl
from jax.experimental.pallas import tpu as pltpu
from jax.experimental.pallas import tpu_sc as plsc

N = 10000
D = 128
HD = D // 2
E = 320000

NUM_CORES = 2
NUM_SUBCORES = 16
NUM_TILES = NUM_CORES * NUM_SUBCORES

N_PAD = 10240
E_PAD = 327680
EDGES_PER_TILE = E_PAD // NUM_TILES
CHUNK = 128
AGGW = 256
AGG_STEPS = EDGES_PER_TILE // AGGW
ROWS_PER_SUBCORE = N_PAD // NUM_SUBCORES

BLK = 2560
GRID = N_PAD // BLK


def _mesh():
    return plsc.VectorSubcoreMesh(core_axis_name="c", subcore_axis_name="s")


def _zero_vmem_2d(ref, rows, cols, dtype=jnp.float32):
    lanes = 32 if dtype == jnp.bfloat16 else 16

    @pl.loop(0, rows)
    def _(r):
        @pl.loop(0, cols, step=lanes)
        def _(j):
            ref[r, pl.ds(j, lanes)] = jnp.zeros((lanes,), dtype)


@jax.jit
def _sc_degree(dst2d, ones_rows):
    @functools.partial(
        pl.kernel,
        out_type=jax.ShapeDtypeStruct((NUM_CORES, N_PAD, 16), jnp.float32),
        mesh=_mesh(),
        compiler_params=pltpu.CompilerParams(use_tc_tiling_on_sc=False),
        scratch_types=[
            pltpu.VMEM((EDGES_PER_TILE,), jnp.int32),
            pltpu.VMEM((AGGW, 16), jnp.float32),
            pltpu.VMEM((CHUNK, 16), jnp.float32),
            pltpu.VMEM_SHARED((N_PAD, 16), jnp.float32),
        ],
    )
    def deg_kernel(dst_hbm, ones_hbm, out_hbm, idx_v, ones_v, zbuf_v, acc_sh):
        c = lax.axis_index("c")
        s = lax.axis_index("s")
        wid = c * NUM_SUBCORES + s

        _zero_vmem_2d(zbuf_v, CHUNK, 16)
        base_row = s * ROWS_PER_SUBCORE

        @pl.loop(0, ROWS_PER_SUBCORE, step=CHUNK)
        def _(j):
            pltpu.sync_copy(zbuf_v, acc_sh.at[pl.ds(base_row + j, CHUNK)])

        pltpu.sync_copy(ones_hbm, ones_v)
        pltpu.sync_copy(
            dst_hbm.at[pl.ds(wid * EDGES_PER_TILE, EDGES_PER_TILE)], idx_v)
        plsc.subcore_barrier()

        @pl.loop(0, AGG_STEPS)
        def _(k):
            pltpu.sync_copy(ones_v, acc_sh.at[idx_v.at[pl.ds(k * AGGW, AGGW)]],
                            add=True)

        plsc.subcore_barrier()
        pltpu.sync_copy(
            acc_sh.at[pl.ds(base_row, ROWS_PER_SUBCORE)],
            out_hbm.at[c, pl.ds(base_row, ROWS_PER_SUBCORE)],
        )

    return deg_kernel(dst2d, ones_rows)


@jax.jit
def _sc_aggregate(y, src2d, dst2d):
    @functools.partial(
        pl.kernel,
        out_type=jax.ShapeDtypeStruct((NUM_CORES, N_PAD, D), jnp.bfloat16),
        mesh=_mesh(),
        compiler_params=pltpu.CompilerParams(use_tc_tiling_on_sc=False),
        scratch_types=[
            pltpu.VMEM((EDGES_PER_TILE,), jnp.int32),
            pltpu.VMEM((EDGES_PER_TILE,), jnp.int32),
            pltpu.VMEM((AGGW, D), jnp.bfloat16),
            pltpu.VMEM((AGGW, D), jnp.bfloat16),
            pltpu.VMEM((CHUNK, D), jnp.bfloat16),
            pltpu.VMEM_SHARED((N_PAD, D), jnp.bfloat16),
            pltpu.SemaphoreType.DMA,
            pltpu.SemaphoreType.DMA,
        ],
    )
    def agg_kernel(y_hbm, src_hbm, dst_hbm, out_hbm,
                   isrc_v, idst_v, rows_a, rows_b, zbuf_v, acc_sh,
                   sem_a, sem_b):
        c = lax.axis_index("c")
        s = lax.axis_index("s")
        wid = c * NUM_SUBCORES + s
        base_row = s * ROWS_PER_SUBCORE

        _zero_vmem_2d(zbuf_v, CHUNK, D, jnp.bfloat16)

        ebase = wid * EDGES_PER_TILE
        pltpu.sync_copy(src_hbm.at[pl.ds(ebase, EDGES_PER_TILE)], isrc_v)
        pltpu.sync_copy(dst_hbm.at[pl.ds(ebase, EDGES_PER_TILE)], idst_v)

        @pl.loop(0, ROWS_PER_SUBCORE, step=CHUNK)
        def _(j):
            pltpu.sync_copy(zbuf_v, acc_sh.at[pl.ds(base_row + j, CHUNK)])

        plsc.subcore_barrier()

        pltpu.make_async_copy(
            y_hbm.at[isrc_v.at[pl.ds(0, AGGW)]], rows_a, sem_a).start()
        pltpu.make_async_copy(
            y_hbm.at[isrc_v.at[pl.ds(AGGW, AGGW)]], rows_b, sem_b).start()

        @pl.loop(0, AGG_STEPS, step=2)
        def _(k):
            for off, buf, sem in ((0, rows_a, sem_a), (1, rows_b, sem_b)):
                kk = (k + off) * AGGW
                pltpu.make_async_copy(
                    y_hbm.at[isrc_v.at[pl.ds(kk, AGGW)]], buf, sem).wait()
                pltpu.sync_copy(buf, acc_sh.at[idst_v.at[pl.ds(kk, AGGW)]],
                                add=True)

                @pl.when(kk + 2 * AGGW < EDGES_PER_TILE)
                def _():
                    pltpu.make_async_copy(
                        y_hbm.at[isrc_v.at[pl.ds(kk + 2 * AGGW, AGGW)]], buf,
                        sem).start()

        plsc.subcore_barrier()
        pltpu.sync_copy(
            acc_sh.at[pl.ds(base_row, ROWS_PER_SUBCORE)],
            out_hbm.at[c, pl.ds(base_row, ROWS_PER_SUBCORE)],
        )

    return agg_kernel(y, src2d, dst2d)


def _dinv_col(deg_ref, blk_i):
    deg = deg_ref[0] + deg_ref[1] + 1.0
    rowid = blk_i * BLK + lax.broadcasted_iota(jnp.int32, (BLK, 16), 0)
    dinv = jnp.where(rowid < N, lax.rsqrt(deg), 0.0)
    return dinv[:, 0:1]


def _xp_gram_body(x_ref, w_ref, xp_ref, g_ref):
    xp = lax.dot_general(x_ref[...], w_ref[...],
                         dimension_numbers=(((1,), (1,)), ((), ())),
                         preferred_element_type=jnp.float32,
                         precision=lax.Precision.DEFAULT)
    xp_ref[...] = xp

    @pl.when(pl.program_id(0) == 0)
    def _():
        g_ref[...] = jnp.zeros((D, D), jnp.float32)

    g_ref[...] += lax.dot_general(xp, xp,
                                  dimension_numbers=(((0,), (0,)), ((), ())),
                                  preferred_element_type=jnp.float32,
                                  precision=lax.Precision.DEFAULT)


@jax.jit
def _tc_xp_gram(x_pad, W1):
    return pl.pallas_call(
        _xp_gram_body,
        grid=(GRID,),
        in_specs=[
            pl.BlockSpec((BLK, D), lambda i: (i, 0)),
            pl.BlockSpec((D, D), lambda i: (0, 0)),
        ],
        out_specs=[
            pl.BlockSpec((BLK, D), lambda i: (i, 0)),
            pl.BlockSpec((D, D), lambda i: (0, 0)),
        ],
        out_shape=[
            jax.ShapeDtypeStruct((N_PAD, D), jnp.float32),
            jax.ShapeDtypeStruct((D, D), jnp.float32),
        ],
    )(x_pad, W1)


def _y_body(xp_ref, deg_ref, y_ref):
    y_ref[...] = (xp_ref[...] * _dinv_col(deg_ref, pl.program_id(0))
                  ).astype(jnp.bfloat16)


@jax.jit
def _tc_y(xp, degp):
    return pl.pallas_call(
        _y_body,
        grid=(GRID,),
        in_specs=[
            pl.BlockSpec((BLK, D), lambda i: (i, 0)),
            pl.BlockSpec((NUM_CORES, BLK, 16), lambda i: (0, i, 0)),
        ],
        out_specs=pl.BlockSpec((BLK, D), lambda i: (i, 0)),
        out_shape=jax.ShapeDtypeStruct((N_PAD, D), jnp.bfloat16),
    )(xp, degp)


def _gram_body(xp_ref, g_ref):
    @pl.when(pl.program_id(0) == 0)
    def _():
        g_ref[...] = jnp.zeros((D, D), jnp.float32)

    xb = xp_ref[...]
    g_ref[...] += lax.dot_general(xb, xb,
                                  dimension_numbers=(((0,), (0,)), ((), ())),
                                  preferred_element_type=jnp.float32,
                                  precision=lax.Precision.DEFAULT)


@jax.jit
def _tc_gram(xp):
    return pl.pallas_call(
        _gram_body,
        grid=(GRID,),
        in_specs=[pl.BlockSpec((BLK, D), lambda i: (i, 0))],
        out_specs=pl.BlockSpec((D, D), lambda i: (0, 0)),
        out_shape=jax.ShapeDtypeStruct((D, D), jnp.float32),
    )(xp)


def _combine_core(xp_ref, y_ref, parts_ref, deg_ref, gmat_ref, gate_ref,
                  b_ref, i):
    zeta, eta, theta = gate_ref[0], gate_ref[1], gate_ref[2]
    xp = xp_ref[...]
    dcol = _dinv_col(deg_ref, i)
    psum = (parts_ref[0].astype(jnp.float32)
            + parts_ref[1].astype(jnp.float32))
    agg = dcol * (psum + y_ref[...].astype(jnp.float32))
    red = lax.dot_general(xp, gmat_ref[...],
                          dimension_numbers=(((1,), (0,)), ((), ())),
                          preferred_element_type=jnp.float32,
                          precision=lax.Precision.DEFAULT)
    return zeta * xp + eta * agg - theta * red + b_ref[...], dcol


def _mid_body(xp_ref, y_ref, parts_ref, deg_ref, gmat_ref, gate_ref, b_ref,
              gam_ref, bet_ref, mu_ref, var_ref, wn_ref, xpn_ref, yn_ref):
    i = pl.program_id(0)
    out, dcol = _combine_core(xp_ref, y_ref, parts_ref, deg_ref, gmat_ref,
                              gate_ref, b_ref, i)
    h = (out - mu_ref[...]) * lax.rsqrt(var_ref[...] + 1e-5) * gam_ref[...] \
        + bet_ref[...]
    h = jnp.maximum(h, 0.0)
    rowid = i * BLK + lax.broadcasted_iota(jnp.int32, (BLK, D), 0)
    h = jnp.where(rowid < N, h, 0.0)
    xpn = lax.dot_general(h, wn_ref[...],
                          dimension_numbers=(((1,), (1,)), ((), ())),
                          preferred_element_type=jnp.float32,
                          precision=lax.Precision.DEFAULT)
    xpn_ref[...] = xpn
    yn_ref[...] = (xpn * dcol).astype(jnp.bfloat16)


@jax.jit
def _tc_mid(xp, y3, parts, degp, gmat, gates, bvec, gam, bet, mu, var, Wn):
    vec = lambda: pl.BlockSpec((1, D), lambda i: (0, 0))
    return pl.pallas_call(
        _mid_body,
        grid=(GRID,),
        in_specs=[
            pl.BlockSpec((BLK, D), lambda i: (i, 0)),
            pl.BlockSpec((BLK, D), lambda i: (i, 0)),
            pl.BlockSpec((NUM_CORES, BLK, D), lambda i: (0, i, 0)),
            pl.BlockSpec((NUM_CORES, BLK, 16), lambda i: (0, i, 0)),
            pl.BlockSpec((D, D), lambda i: (0, 0)),
            pl.BlockSpec(memory_space=pltpu.SMEM),
            vec(), vec(), vec(), vec(), vec(),
            pl.BlockSpec((D, D), lambda i: (0, 0)),
        ],
        out_specs=[
            pl.BlockSpec((BLK, D), lambda i: (i, 0)),
            pl.BlockSpec((BLK, D), lambda i: (i, 0)),
        ],
        out_shape=[
            jax.ShapeDtypeStruct((N_PAD, D), jnp.float32),
            jax.ShapeDtypeStruct((N_PAD, D), jnp.bfloat16),
        ],
    )(xp, y3, parts, degp, gmat, gates, bvec, gam, bet, mu, var, Wn)


def _last_body(xp_ref, y_ref, parts_ref, deg_ref, gmat_ref, gate_ref, b_ref,
               out_ref):
    i = pl.program_id(0)
    out, _ = _combine_core(xp_ref, y_ref, parts_ref, deg_ref, gmat_ref,
                           gate_ref, b_ref, i)
    out_ref[...] = out


@jax.jit
def _tc_last(xp, y3, parts, degp, gmat, gates, bvec):
    return pl.pallas_call(
        _last_body,
        grid=(GRID,),
        in_specs=[
            pl.BlockSpec((BLK, D), lambda i: (i, 0)),
            pl.BlockSpec((BLK, D), lambda i: (i, 0)),
            pl.BlockSpec((NUM_CORES, BLK, D), lambda i: (0, i, 0)),
            pl.BlockSpec((NUM_CORES, BLK, 16), lambda i: (0, i, 0)),
            pl.BlockSpec((D, D), lambda i: (0, 0)),
            pl.BlockSpec(memory_space=pltpu.SMEM),
            pl.BlockSpec((1, D), lambda i: (0, 0)),
        ],
        out_specs=pl.BlockSpec((BLK, D), lambda i: (i, 0)),
        out_shape=jax.ShapeDtypeStruct((N_PAD, D), jnp.float32),
    )(xp, y3, parts, degp, gmat, gates, bvec)


def kernel(x, edge_index, W1, b1, g1, W2, b2, g2, W3, b3, g3,
           bn1_gamma, bn1_beta, bn1_mean, bn1_var,
           bn2_gamma, bn2_beta, bn2_mean, bn2_var):
    pad_idx = N + (jnp.arange(E_PAD - E, dtype=jnp.int32) % (N_PAD - N))
    src_flat = jnp.concatenate([edge_index[0], pad_idx])
    dst_flat = jnp.concatenate([edge_index[1], pad_idx])
    x_pad = jnp.concatenate(
        [x, jnp.zeros((N_PAD - N, D), dtype=jnp.float32)], axis=0)
    ones_rows = jnp.ones((AGGW, 16), jnp.float32)

    degp = _sc_degree(dst_flat, ones_rows)
    xp, gmat1 = _tc_xp_gram(x_pad, W1)
    y3 = _tc_y(xp, degp)

    bn = [(bn1_gamma, bn1_beta, bn1_mean, bn1_var),
          (bn2_gamma, bn2_beta, bn2_mean, bn2_var)]
    row = lambda v: v.reshape(1, D)

    for lyr, (gate, bvec, Wn) in enumerate(((g1, b1, W2), (g2, b2, W3))):
        parts = _sc_aggregate(y3, src_flat, dst_flat)
        gmat = gmat1 if lyr == 0 else _tc_gram(xp)
        gam, bet, mu, var = bn[lyr]
        xp, y3 = _tc_mid(xp, y3, parts, degp, gmat, jax.nn.softmax(gate),
                         row(bvec), row(gam), row(bet), row(mu), row(var), Wn)

    parts = _sc_aggregate(y3, src_flat, dst_flat)
    gmat = _tc_gram(xp)
    out = _tc_last(xp, y3, parts, degp, gmat, jax.nn.softmax(g3), row(b3))
    return out[:N]
